# Initial kernel scaffold; baseline (speedup 1.0000x reference)
import jax
import jax.numpy as jnp
from jax.experimental import pallas as pl


def kernel(token_ids, embedding_weight):
    raise NotImplementedError("write your pallas kernel here")


# speedup vs baseline: 1.3062x; 1.3062x over previous
import functools
import math

import jax
import jax.numpy as jnp
from jax import lax
from jax.experimental import pallas as pl
from jax.experimental.pallas import tpu as pltpu
from jax.experimental.pallas import tpu_sc as plsc


def _make_emb_kernel(B, D, C, NC, NS):
    NW = NC * NS
    b_per_w = B // NW
    n_chunks = b_per_w // C
    mesh = plsc.VectorSubcoreMesh(core_axis_name="c", subcore_axis_name="s")
    scale = math.sqrt(D)
    lanes_per_row = D // 16

    @functools.partial(
        pl.kernel,
        mesh=mesh,
        compiler_params=pltpu.CompilerParams(use_tc_tiling_on_sc=False),
        out_type=jax.ShapeDtypeStruct((B, D), jnp.float32),
        scratch_types=[
            pltpu.VMEM((C,), jnp.int32),
            pltpu.VMEM((C, D), jnp.float32),
            pltpu.SemaphoreType.DMA,
        ],
    )
    def emb(ids_hbm, table_hbm, out_hbm, idx_v, rows_v, sem):
        wid = lax.axis_index("s") * NC + lax.axis_index("c")
        base = wid * b_per_w

        def chunk_body(g, _):
            off = base + g * C
            pltpu.sync_copy(ids_hbm.at[pl.ds(off, C)], idx_v)
            pltpu.async_copy(table_hbm.at[idx_v], rows_v, sem).wait()

            def scale_row(r, _):
                for j in range(lanes_per_row):
                    sl = pl.ds(j * 16, 16)
                    rows_v[r, sl] = rows_v[r, sl] * scale
                return 0

            lax.fori_loop(0, C, scale_row, 0)
            pltpu.sync_copy(rows_v, out_hbm.at[pl.ds(off, C)])
            return 0

        lax.fori_loop(0, n_chunks, chunk_body, 0)

    return emb


def kernel(token_ids, embedding_weight):
    S1, S2 = token_ids.shape
    V, D = embedding_weight.shape
    B = S1 * S2
    info = plsc.get_sparse_core_info()
    NC, NS = info.num_cores, info.num_subcores
    C = 1600
    flat_ids = token_ids.reshape(B).astype(jnp.int32)
    emb = _make_emb_kernel(B, D, C, NC, NS)
    out = emb(flat_ids, embedding_weight)
    return out.reshape(S1, S2, D)


# speedup vs baseline: 1.4816x; 1.1344x over previous
import functools
import math

import jax
import jax.numpy as jnp
from jax import lax
from jax.experimental import pallas as pl
from jax.experimental.pallas import tpu as pltpu
from jax.experimental.pallas import tpu_sc as plsc

_NBUF = 4


def _make_emb_kernel(B, D, C, NC, NS):
    NW = NC * NS
    b_per_w = B // NW
    n_chunks = b_per_w // C
    assert n_chunks % _NBUF == 0 and n_chunks >= 2 * _NBUF
    mesh = plsc.VectorSubcoreMesh(core_axis_name="c", subcore_axis_name="s")
    scale = math.sqrt(D)
    lanes_per_row = D // 16

    @functools.partial(
        pl.kernel,
        mesh=mesh,
        compiler_params=pltpu.CompilerParams(use_tc_tiling_on_sc=False),
        out_type=jax.ShapeDtypeStruct((B, D), jnp.float32),
        scratch_types=[
            pltpu.VMEM((n_chunks, C), jnp.int32),
            pltpu.VMEM((_NBUF, C, D), jnp.float32),
            [pltpu.SemaphoreType.DMA] * _NBUF,
            [pltpu.SemaphoreType.DMA] * _NBUF,
        ],
    )
    def emb(ids_hbm, table_hbm, out_hbm, idx_v, rows_v, gsem, wsem):
        wid = lax.axis_index("s") * NC + lax.axis_index("c")
        base = wid * b_per_w

        pltpu.sync_copy(ids_hbm.at[wid], idx_v)

        def issue_gather(g, b):
            pltpu.async_copy(table_hbm.at[idx_v.at[g]], rows_v.at[b], gsem[b])

        def wait_gather(b):
            pltpu.make_async_copy(
                table_hbm.at[idx_v.at[0]], rows_v.at[b], gsem[b]
            ).wait()

        def issue_writeback(g, b):
            pltpu.async_copy(rows_v.at[b], out_hbm.at[pl.ds(base + g * C, C)], wsem[b])

        def wait_writeback(b):
            pltpu.make_async_copy(
                rows_v.at[b], out_hbm.at[pl.ds(base, C)], wsem[b]
            ).wait()

        def scale_buf(b):
            R = 8

            def scale_rows(i, _):
                r0 = i * R
                for rr in range(R):
                    for j in range(lanes_per_row):
                        sl = pl.ds(j * 16, 16)
                        rows_v[b, r0 + rr, sl] = rows_v[b, r0 + rr, sl] * scale
                return 0

            lax.fori_loop(0, C // R, scale_rows, 0)

        issue_gather(0, 0)
        issue_gather(1, 1)

        @pl.loop(0, n_chunks, step=_NBUF)
        def _(go):
            for b in range(_NBUF):
                g = go + b
                wait_gather(b)
                b2 = (b + 2) % _NBUF

                @pl.when(g + 2 < n_chunks)
                def _():
                    if b >= 2:
                        wait_writeback(b2)
                    else:
                        @pl.when(go > 0)
                        def _():
                            wait_writeback(b2)

                    issue_gather(g + 2, b2)

                scale_buf(b)
                issue_writeback(g, b)

        for b in range(_NBUF):
            wait_writeback(b)

    return emb


def kernel(token_ids, embedding_weight):
    S1, S2 = token_ids.shape
    V, D = embedding_weight.shape
    B = S1 * S2
    info = plsc.get_sparse_core_info()
    NC, NS = info.num_cores, info.num_subcores
    NW = NC * NS
    C = 800
    b_per_w = B // NW
    n_chunks = b_per_w // C
    flat_ids = token_ids.reshape(NW, n_chunks, C).astype(jnp.int32)
    emb = _make_emb_kernel(B, D, C, NC, NS)
    out = emb(flat_ids, embedding_weight)
    return out.reshape(S1, S2, D)
